# baseline (speedup 1.0000x reference)
import jax
import jax.numpy as jnp
from jax.experimental import pallas as pl
from jax.experimental.pallas import tpu as pltpu

DIM = 128


def _make_body(chunk_rows, chunk_offs):
    n_chunks = len(chunk_rows)

    def body(x_hbm, o_hbm, *bufs_and_sems):
        bufs = bufs_and_sems[:n_chunks]
        in_sems, out_sems = bufs_and_sems[n_chunks], bufs_and_sems[n_chunks + 1]

        def read(i):
            return pltpu.make_async_copy(
                x_hbm.at[pl.ds(chunk_offs[i], chunk_rows[i])], bufs[i],
                in_sems.at[i],
            )

        def write(i):
            return pltpu.make_async_copy(
                bufs[i], o_hbm.at[pl.ds(chunk_offs[i], chunk_rows[i])],
                out_sems.at[i],
            )

        for i in range(n_chunks):
            read(i).start()
        for i in range(n_chunks):
            read(i).wait()
            write(i).start()
        for i in range(n_chunks):
            write(i).wait()

    return body


def kernel(x):
    x = x.astype(jnp.float32)
    n = x.size // DIM
    x = x.reshape(n, DIM)
    q = (n // 4) // 8 * 8
    chunk_rows = (q, q, q, n - 3 * q) if 0 < q and 3 * q < n else (n,)
    chunk_offs = tuple(sum(chunk_rows[:i]) for i in range(len(chunk_rows)))
    return pl.pallas_call(
        _make_body(chunk_rows, chunk_offs),
        out_shape=jax.ShapeDtypeStruct((n, DIM), jnp.float32),
        in_specs=[pl.BlockSpec(memory_space=pltpu.MemorySpace.HBM)],
        out_specs=pl.BlockSpec(memory_space=pltpu.MemorySpace.HBM),
        scratch_shapes=[
            *[pltpu.VMEM((r, DIM), jnp.float32) for r in chunk_rows],
            pltpu.SemaphoreType.DMA((len(chunk_rows),)),
            pltpu.SemaphoreType.DMA((len(chunk_rows),)),
        ],
    )(x)


# speedup vs baseline: 1.0166x; 1.0166x over previous
import jax
import jax.numpy as jnp
from jax.experimental import pallas as pl
from jax.experimental.pallas import tpu as pltpu

DIM = 128


def _make_body(chunk_rows, chunk_offs):
    n_chunks = len(chunk_rows)

    def body(x_hbm, o_hbm, *bufs_and_sems):
        bufs = bufs_and_sems[:n_chunks]
        in_sems, out_sems = bufs_and_sems[n_chunks], bufs_and_sems[n_chunks + 1]

        def read(i):
            return pltpu.make_async_copy(
                x_hbm.at[pl.ds(chunk_offs[i], chunk_rows[i])], bufs[i],
                in_sems.at[i],
            )

        def write(i):
            return pltpu.make_async_copy(
                bufs[i], o_hbm.at[pl.ds(chunk_offs[i], chunk_rows[i])],
                out_sems.at[i],
            )

        for i in range(n_chunks):
            read(i).start()
        for i in range(n_chunks):
            read(i).wait()
            write(i).start()
        for i in range(n_chunks):
            write(i).wait()

    return body


def kernel(x):
    x = x.astype(jnp.float32)
    n = x.size // DIM
    x = x.reshape(n, DIM)
    half = (n // 2) // 8 * 8
    chunk_rows = (half, n - half) if 0 < half < n else (n,)
    chunk_offs = tuple(sum(chunk_rows[:i]) for i in range(len(chunk_rows)))
    return pl.pallas_call(
        _make_body(chunk_rows, chunk_offs),
        out_shape=jax.ShapeDtypeStruct((n, DIM), jnp.float32),
        in_specs=[pl.BlockSpec(memory_space=pltpu.MemorySpace.HBM)],
        out_specs=pl.BlockSpec(memory_space=pltpu.MemorySpace.HBM),
        scratch_shapes=[
            *[pltpu.VMEM((r, DIM), jnp.float32) for r in chunk_rows],
            pltpu.SemaphoreType.DMA((len(chunk_rows),)),
            pltpu.SemaphoreType.DMA((len(chunk_rows),)),
        ],
    )(x)
